# baseline (speedup 1.0000x reference)
import jax
import jax.numpy as jnp
from jax.experimental import pallas as pl
from jax.experimental.pallas import tpu as pltpu


_BLOCK_ROWS = 2048


def _copy_kernel(src_ref, dst_ref):
    dst_ref[...] = src_ref[...]


def kernel(prototypes):
    rows, feat = prototypes.shape
    n = rows // _BLOCK_ROWS
    return pl.pallas_call(
        _copy_kernel,
        out_shape=jax.ShapeDtypeStruct(prototypes.shape, prototypes.dtype),
        grid=(2, n // 2),
        in_specs=[
            pl.BlockSpec((_BLOCK_ROWS, feat), lambda c, i: (c * (n // 2) + i, 0))
        ],
        out_specs=pl.BlockSpec(
            (_BLOCK_ROWS, feat), lambda c, i: (c * (n // 2) + i, 0)
        ),
        compiler_params=pltpu.CompilerParams(
            dimension_semantics=("parallel", "arbitrary")
        ),
    )(prototypes)
